# baseline (speedup 1.0000x reference)
import functools

import jax
import jax.numpy as jnp
from jax import lax
from jax.experimental import pallas as pl
from jax.experimental.pallas import tpu as pltpu
from jax.experimental.pallas import tpu_sc as plsc

B = 4
S = 2048
T = B * S
D = 1024
H = 4096
E = 8
CAP = int(1.25 * T / E)
NSLOT = E * CAP
PAD_ROW = NSLOT
AUX_COEF = 0.01

RT = 8
RB = T // RT

MB = 256
MR = CAP // MB
NH = 2
HB = H // NH


def _router_body(x_ref, rw_ref, rb_ref, ti_ref, tv_ref, cs_ref, cnt_ref,
                 aux_ref):
    i = pl.program_id(0)
    xb = x_ref[...]
    logits = jnp.dot(xb, rw_ref[...], preferred_element_type=jnp.float32)
    logits = logits + rb_ref[...]
    m = jnp.max(logits, axis=-1, keepdims=True)
    ex = jnp.exp(logits - m)
    gates = ex / jnp.sum(ex, axis=-1, keepdims=True)
    tv = jnp.max(gates, axis=-1)
    ti = jnp.argmax(gates, axis=-1).astype(jnp.int32)
    ti_ref[...] = ti.reshape(1, 1, RB)
    tv_ref[...] = tv.reshape(1, 1, RB)
    cs = jnp.sum(gates, axis=0).reshape(1, E)
    oh = (ti[:, None] == lax.broadcasted_iota(jnp.int32, (1, E), 1))
    cnt = jnp.sum(oh.astype(jnp.float32), axis=0).reshape(1, E)

    @pl.when(i == 0)
    def _():
        cs_ref[...] = cs
        cnt_ref[...] = cnt

    @pl.when(i > 0)
    def _():
        cs_ref[...] = cs_ref[...] + cs
        cnt_ref[...] = cnt_ref[...] + cnt

    @pl.when(i == RT - 1)
    def _():
        imp = cs_ref[...] / float(T)
        load = cnt_ref[...] / float(T)
        aux_ref[...] = jnp.sum(imp * load).reshape(1, 1) * (E * AUX_COEF)


def _router(x_flat, rw, rb2):
    return pl.pallas_call(
        _router_body,
        grid=(RT,),
        in_specs=[
            pl.BlockSpec((RB, D), lambda i: (i, 0)),
            pl.BlockSpec((D, E), lambda i: (0, 0)),
            pl.BlockSpec((1, E), lambda i: (0, 0)),
        ],
        out_specs=[
            pl.BlockSpec((1, 1, RB), lambda i: (i, 0, 0)),
            pl.BlockSpec((1, 1, RB), lambda i: (i, 0, 0)),
            pl.BlockSpec((1, E), lambda i: (0, 0)),
            pl.BlockSpec((1, E), lambda i: (0, 0)),
            pl.BlockSpec((1, 1), lambda i: (0, 0)),
        ],
        out_shape=[
            jax.ShapeDtypeStruct((RT, 1, RB), jnp.int32),
            jax.ShapeDtypeStruct((RT, 1, RB), jnp.float32),
            jax.ShapeDtypeStruct((1, E), jnp.float32),
            jax.ShapeDtypeStruct((1, E), jnp.float32),
            jax.ShapeDtypeStruct((1, 1), jnp.float32),
        ],
    )(x_flat, rw, rb2)


def _dispatch_body(ti_hbm, tv_hbm, disp_hbm, gate_hbm, cnt_hbm, comb_hbm,
                   idx_v, tv_v, inv_v, disp_v, gate_v, s16_v, buf_v, acc_v,
                   inv_sh):
    c = lax.axis_index("c")
    s = lax.axis_index("s")

    @pl.when(jnp.logical_and(c == 0, s < E))
    def _phase_a():
        e = s
        pltpu.sync_copy(ti_hbm, idx_v)
        pltpu.sync_copy(tv_hbm, tv_v)

        def tok_body(i, base):
            v = idx_v[pl.ds(i * 16, 16)]
            m = v == e
            mi = jnp.where(m, 1, 0)
            csum = plsc.cumsum(mi)
            ranks = base + csum - 1
            keep = jnp.logical_and(m, ranks < CAP)
            tok = lax.iota(jnp.int32, 16) + i * 16
            slotp1 = e * CAP + ranks + 1
            inv_v[pl.ds(i * 16, 16)] = jnp.where(keep, slotp1, 0)
            ranks_c = jnp.where(keep, ranks, 0)
            plsc.store_scatter(disp_v, [ranks_c], tok, mask=keep)
            return base + jnp.sum(mi)

        total = lax.fori_loop(0, T // 16, tok_body, jnp.int32(0))
        count = jnp.minimum(total, CAP)

        def gate_body(j, _):
            pos = lax.iota(jnp.int32, 16) + j * 16
            valid = pos < count
            idxs = jnp.where(valid, disp_v[pl.ds(j * 16, 16)], 0)
            g = plsc.load_gather(tv_v, [idxs])
            gate_v[pl.ds(j * 16, 16)] = jnp.where(valid, g, 0.0)
            disp_v[pl.ds(j * 16, 16)] = idxs
            return 0

        lax.fori_loop(0, CAP // 16, gate_body, 0)
        s16_v[...] = jnp.where(lax.iota(jnp.int32, 16) == 0, count, 0)
        pltpu.sync_copy(disp_v, disp_hbm.at[e])
        pltpu.sync_copy(gate_v, gate_hbm.at[e])
        pltpu.sync_copy(s16_v, cnt_hbm.at[e])
        pltpu.sync_copy(inv_v, inv_sh.at[e])

    plsc.subcore_barrier()

    @pl.when(c == 0)
    def _phase_b():
        t0 = s * (T // 16)
        nch = T // 16 // 16
        for e in range(E):
            pltpu.sync_copy(inv_sh.at[e, pl.ds(t0, T // 16)], buf_v)

            def add_body(k, _):
                chunk = buf_v[pl.ds(k * 16, 16)]
                if e == 0:
                    acc_v[pl.ds(k * 16, 16)] = chunk
                else:
                    acc_v[pl.ds(k * 16, 16)] = acc_v[pl.ds(k * 16, 16)] + chunk
                return 0

            lax.fori_loop(0, nch, add_body, 0)

        def fin_body(k, _):
            iv = acc_v[pl.ds(k * 16, 16)]
            buf_v[pl.ds(k * 16, 16)] = jnp.where(iv > 0, iv - 1, PAD_ROW)
            return 0

        lax.fori_loop(0, nch, fin_body, 0)
        pltpu.sync_copy(buf_v, comb_hbm.at[pl.ds(t0, T // 16)])


def _dispatch(top_idx, top_vals):
    mesh = plsc.VectorSubcoreMesh(core_axis_name="c", subcore_axis_name="s")
    fn = pl.kernel(
        _dispatch_body,
        mesh=mesh,
        compiler_params=pltpu.CompilerParams(needs_layout_passes=False),
        out_type=[
            jax.ShapeDtypeStruct((E, CAP), jnp.int32),
            jax.ShapeDtypeStruct((E, CAP), jnp.float32),
            jax.ShapeDtypeStruct((E, 16), jnp.int32),
            jax.ShapeDtypeStruct((T,), jnp.int32),
        ],
        scratch_types=[
            pltpu.VMEM((T,), jnp.int32),
            pltpu.VMEM((T,), jnp.float32),
            pltpu.VMEM((T,), jnp.int32),
            pltpu.VMEM((CAP,), jnp.int32),
            pltpu.VMEM((CAP,), jnp.float32),
            pltpu.VMEM((16,), jnp.int32),
            pltpu.VMEM((T // 16,), jnp.int32),
            pltpu.VMEM((T // 16,), jnp.int32),
            pltpu.MemorySpace.VMEM_SHARED((E, T), jnp.int32),
        ],
    )
    return fn(top_idx, top_vals)


def _row_gather_body(nrows, chunk, tbl_hbm, idx_hbm, out_hbm, idx_v, buf_v,
                     sem):
    c = lax.axis_index("c")
    s = lax.axis_index("s")
    wid = s * 2 + c
    base = wid * nrows
    pltpu.sync_copy(idx_hbm.at[pl.ds(base, nrows)], idx_v)

    def body(j, _):
        cp = pltpu.async_copy(tbl_hbm.at[idx_v.at[pl.ds(j * chunk, chunk)]],
                              buf_v, sem)
        cp.wait()
        pltpu.sync_copy(buf_v, out_hbm.at[pl.ds(base + j * chunk, chunk)])
        return 0

    lax.fori_loop(0, nrows // chunk, body, 0)


def _row_gather(tbl, idx, nrows_total):
    nrows = nrows_total // 32
    chunk = 16
    mesh = plsc.VectorSubcoreMesh(core_axis_name="c", subcore_axis_name="s")
    fn = pl.kernel(
        functools.partial(_row_gather_body, nrows, chunk),
        mesh=mesh,
        out_type=jax.ShapeDtypeStruct((nrows_total, D), jnp.float32),
        scratch_types=[
            pltpu.VMEM((nrows,), jnp.int32),
            pltpu.VMEM((chunk, D), jnp.float32),
            pltpu.SemaphoreType.DMA,
        ],
    )
    return fn(tbl, idx)


def _mlp_body(xe_ref, w1_ref, b1_ref, w2_ref, b2_ref, g_ref, out_ref,
              acc_ref, w1b_ref, w2b_ref):
    h = pl.program_id(1)
    r = pl.program_id(2)

    @pl.when(r == 0)
    def _():
        w1b_ref[...] = w1_ref[0].astype(jnp.bfloat16)
        w2b_ref[...] = w2_ref[0].astype(jnp.bfloat16)

    xb = xe_ref[...].astype(jnp.bfloat16)
    hid = jnp.dot(xb, w1b_ref[...], preferred_element_type=jnp.float32)
    hid = jnp.maximum(hid + b1_ref[0], 0.0)
    partial = jnp.dot(hid.astype(jnp.bfloat16), w2b_ref[...],
                      preferred_element_type=jnp.float32)

    @pl.when(h == 0)
    def _():
        acc_ref[pl.ds(r * MB, MB), :] = partial

    @pl.when(h == NH - 1)
    def _():
        o = acc_ref[pl.ds(r * MB, MB), :] + partial + b2_ref[0]
        out_ref[...] = o * g_ref[0, 0].reshape(MB, 1)


def _mlp(xe, w1, b1r, w2, b2r, gate3):
    return pl.pallas_call(
        _mlp_body,
        grid=(E, NH, MR),
        in_specs=[
            pl.BlockSpec((MB, D), lambda e, h, r: (e * MR + r, 0)),
            pl.BlockSpec((1, D, HB), lambda e, h, r: (e, 0, h)),
            pl.BlockSpec((1, 1, HB), lambda e, h, r: (e, 0, h)),
            pl.BlockSpec((1, HB, D), lambda e, h, r: (e, h, 0)),
            pl.BlockSpec((1, 1, D), lambda e, h, r: (e, 0, 0)),
            pl.BlockSpec((1, 1, MB), lambda e, h, r: (e * MR + r, 0, 0)),
        ],
        out_specs=pl.BlockSpec((MB, D), lambda e, h, r: (e * MR + r, 0)),
        out_shape=jax.ShapeDtypeStruct((NSLOT, D), jnp.float32),
        scratch_shapes=[
            pltpu.VMEM((CAP, D), jnp.float32),
            pltpu.VMEM((D, HB), jnp.bfloat16),
            pltpu.VMEM((HB, D), jnp.bfloat16),
        ],
    )(xe, w1, b1r, w2, b2r, gate3)


def kernel(x, router_w, router_b, w1, b1, w2, b2):
    x_flat = x.reshape(T, D)
    ti3, tv3, _cs, _cnt, aux = _router(x_flat, router_w,
                                       router_b.reshape(1, E))
    top_idx = ti3.reshape(T)
    top_vals = tv3.reshape(T)
    disp, gate, cnt16, comb_idx = _dispatch(top_idx, top_vals)
    xe = _row_gather(x_flat, disp.reshape(NSLOT), NSLOT)
    out_all = _mlp(xe, w1, b1.reshape(E, 1, H), w2, b2.reshape(E, 1, D),
                   gate.reshape(E * MR, 1, MB))
    out_pad = jnp.concatenate(
        [out_all, jnp.zeros((8, D), jnp.float32)], axis=0)
    y_flat = _row_gather(out_pad, comb_idx, T)
    return (y_flat.reshape(B, S, D), aux.reshape(()), cnt16[:, 0])


# speedup vs baseline: 1.0398x; 1.0398x over previous
import functools

import jax
import jax.numpy as jnp
from jax import lax
from jax.experimental import pallas as pl
from jax.experimental.pallas import tpu as pltpu
from jax.experimental.pallas import tpu_sc as plsc

B = 4
S = 2048
T = B * S
D = 1024
H = 4096
E = 8
CAP = int(1.25 * T / E)
NSLOT = E * CAP
PAD_ROW = NSLOT
AUX_COEF = 0.01

RT = 8
RB = T // RT

MB = 256
MR = CAP // MB
NH = 2
HB = H // NH


def _router_body(x_ref, rw_ref, rb_ref, ti_ref, tv_ref, cs_ref, cnt_ref,
                 aux_ref):
    i = pl.program_id(0)
    xb = x_ref[...]
    logits = jnp.dot(xb, rw_ref[...], preferred_element_type=jnp.float32)
    logits = logits + rb_ref[...]
    m = jnp.max(logits, axis=-1, keepdims=True)
    ex = jnp.exp(logits - m)
    gates = ex / jnp.sum(ex, axis=-1, keepdims=True)
    tv = jnp.max(gates, axis=-1)
    ti = jnp.argmax(gates, axis=-1).astype(jnp.int32)
    ti_ref[...] = ti.reshape(1, 1, RB)
    tv_ref[...] = tv.reshape(1, 1, RB)
    cs = jnp.sum(gates, axis=0).reshape(1, E)
    oh = (ti[:, None] == lax.broadcasted_iota(jnp.int32, (1, E), 1))
    cnt = jnp.sum(oh.astype(jnp.float32), axis=0).reshape(1, E)

    @pl.when(i == 0)
    def _():
        cs_ref[...] = cs
        cnt_ref[...] = cnt

    @pl.when(i > 0)
    def _():
        cs_ref[...] = cs_ref[...] + cs
        cnt_ref[...] = cnt_ref[...] + cnt

    @pl.when(i == RT - 1)
    def _():
        imp = cs_ref[...] / float(T)
        load = cnt_ref[...] / float(T)
        aux_ref[...] = jnp.sum(imp * load).reshape(1, 1) * (E * AUX_COEF)


def _router(x_flat, rw, rb2):
    return pl.pallas_call(
        _router_body,
        grid=(RT,),
        in_specs=[
            pl.BlockSpec((RB, D), lambda i: (i, 0)),
            pl.BlockSpec((D, E), lambda i: (0, 0)),
            pl.BlockSpec((1, E), lambda i: (0, 0)),
        ],
        out_specs=[
            pl.BlockSpec((1, 1, RB), lambda i: (i, 0, 0)),
            pl.BlockSpec((1, 1, RB), lambda i: (i, 0, 0)),
            pl.BlockSpec((1, E), lambda i: (0, 0)),
            pl.BlockSpec((1, E), lambda i: (0, 0)),
            pl.BlockSpec((1, 1), lambda i: (0, 0)),
        ],
        out_shape=[
            jax.ShapeDtypeStruct((RT, 1, RB), jnp.int32),
            jax.ShapeDtypeStruct((RT, 1, RB), jnp.float32),
            jax.ShapeDtypeStruct((1, E), jnp.float32),
            jax.ShapeDtypeStruct((1, E), jnp.float32),
            jax.ShapeDtypeStruct((1, 1), jnp.float32),
        ],
    )(x_flat, rw, rb2)


def _dispatch_body(ti_hbm, tv_hbm, disp_hbm, gate_hbm, cnt_hbm, comb_hbm,
                   idx_v, tv_v, inv_v, disp_v, gate_v, s16_v, buf_v, acc_v,
                   inv_sh):
    c = lax.axis_index("c")
    s = lax.axis_index("s")

    @pl.when(jnp.logical_and(c == 0, s < E))
    def _phase_a():
        e = s
        pltpu.sync_copy(ti_hbm, idx_v)
        pltpu.sync_copy(tv_hbm, tv_v)

        def tok_body(i, base):
            v = idx_v[pl.ds(i * 16, 16)]
            m = v == e
            mi = jnp.where(m, 1, 0)
            csum = plsc.cumsum(mi)
            ranks = base + csum - 1
            keep = jnp.logical_and(m, ranks < CAP)
            tok = lax.iota(jnp.int32, 16) + i * 16
            slotp1 = e * CAP + ranks + 1
            inv_v[pl.ds(i * 16, 16)] = jnp.where(keep, slotp1, 0)
            ranks_c = jnp.where(keep, ranks, 0)
            plsc.store_scatter(disp_v, [ranks_c], tok, mask=keep)
            return base + jnp.sum(mi)

        total = lax.fori_loop(0, T // 16, tok_body, jnp.int32(0))
        count = jnp.minimum(total, CAP)

        def gate_body(j, _):
            pos = lax.iota(jnp.int32, 16) + j * 16
            valid = pos < count
            idxs = jnp.where(valid, disp_v[pl.ds(j * 16, 16)], 0)
            g = plsc.load_gather(tv_v, [idxs])
            gate_v[pl.ds(j * 16, 16)] = jnp.where(valid, g, 0.0)
            disp_v[pl.ds(j * 16, 16)] = idxs
            return 0

        lax.fori_loop(0, CAP // 16, gate_body, 0)
        s16_v[...] = jnp.where(lax.iota(jnp.int32, 16) == 0, count, 0)
        pltpu.sync_copy(disp_v, disp_hbm.at[e])
        pltpu.sync_copy(gate_v, gate_hbm.at[e])
        pltpu.sync_copy(s16_v, cnt_hbm.at[e])
        pltpu.sync_copy(inv_v, inv_sh.at[e])

    plsc.subcore_barrier()

    @pl.when(c == 0)
    def _phase_b():
        t0 = s * (T // 16)
        nch = T // 16 // 16
        for e in range(E):
            pltpu.sync_copy(inv_sh.at[e, pl.ds(t0, T // 16)], buf_v)

            def add_body(k, _):
                chunk = buf_v[pl.ds(k * 16, 16)]
                if e == 0:
                    acc_v[pl.ds(k * 16, 16)] = chunk
                else:
                    acc_v[pl.ds(k * 16, 16)] = acc_v[pl.ds(k * 16, 16)] + chunk
                return 0

            lax.fori_loop(0, nch, add_body, 0)

        def fin_body(k, _):
            iv = acc_v[pl.ds(k * 16, 16)]
            buf_v[pl.ds(k * 16, 16)] = jnp.where(iv > 0, iv - 1, PAD_ROW)
            return 0

        lax.fori_loop(0, nch, fin_body, 0)
        pltpu.sync_copy(buf_v, comb_hbm.at[pl.ds(t0, T // 16)])


def _dispatch(top_idx, top_vals):
    mesh = plsc.VectorSubcoreMesh(core_axis_name="c", subcore_axis_name="s")
    fn = pl.kernel(
        _dispatch_body,
        mesh=mesh,
        compiler_params=pltpu.CompilerParams(needs_layout_passes=False),
        out_type=[
            jax.ShapeDtypeStruct((E, CAP), jnp.int32),
            jax.ShapeDtypeStruct((E, CAP), jnp.float32),
            jax.ShapeDtypeStruct((E, 16), jnp.int32),
            jax.ShapeDtypeStruct((T,), jnp.int32),
        ],
        scratch_types=[
            pltpu.VMEM((T,), jnp.int32),
            pltpu.VMEM((T,), jnp.float32),
            pltpu.VMEM((T,), jnp.int32),
            pltpu.VMEM((CAP,), jnp.int32),
            pltpu.VMEM((CAP,), jnp.float32),
            pltpu.VMEM((16,), jnp.int32),
            pltpu.VMEM((T // 16,), jnp.int32),
            pltpu.VMEM((T // 16,), jnp.int32),
            pltpu.MemorySpace.VMEM_SHARED((E, T), jnp.int32),
        ],
    )
    return fn(top_idx, top_vals)


def _row_gather_body(nrows, chunk, tbl_hbm, idx_hbm, out_hbm, idx_v, buf0,
                     buf1, gs0, gs1, ws0, ws1):
    c = lax.axis_index("c")
    s = lax.axis_index("s")
    wid = s * 2 + c
    base = wid * nrows
    pltpu.sync_copy(idx_hbm.at[pl.ds(base, nrows)], idx_v)
    n = nrows // chunk
    bufs = (buf0, buf1)
    gsems = (gs0, gs1)
    wsems = (ws0, ws1)

    def start_gather(j):
        return pltpu.async_copy(
            tbl_hbm.at[idx_v.at[pl.ds(j * chunk, chunk)]], bufs[j % 2],
            gsems[j % 2])

    def start_write(j):
        return pltpu.async_copy(
            bufs[j % 2], out_hbm.at[pl.ds(base + j * chunk, chunk)],
            wsems[j % 2])

    gh = {0: start_gather(0)}
    wh = {}
    for j in range(n):
        if j + 1 < n:
            if j - 1 in wh:
                wh[j - 1].wait()
            gh[j + 1] = start_gather(j + 1)
        gh[j].wait()
        wh[j] = start_write(j)
    wh[n - 2].wait()
    wh[n - 1].wait()


def _row_gather(tbl, idx, nrows_total):
    nrows = nrows_total // 32
    chunk = 32
    mesh = plsc.VectorSubcoreMesh(core_axis_name="c", subcore_axis_name="s")
    fn = pl.kernel(
        functools.partial(_row_gather_body, nrows, chunk),
        mesh=mesh,
        compiler_params=pltpu.CompilerParams(needs_layout_passes=False),
        out_type=jax.ShapeDtypeStruct((nrows_total, D), jnp.float32),
        scratch_types=[
            pltpu.VMEM((nrows,), jnp.int32),
            pltpu.VMEM((chunk, D), jnp.float32),
            pltpu.VMEM((chunk, D), jnp.float32),
            pltpu.SemaphoreType.DMA,
            pltpu.SemaphoreType.DMA,
            pltpu.SemaphoreType.DMA,
            pltpu.SemaphoreType.DMA,
        ],
    )
    return fn(tbl, idx)


def _mlp_body(xe_ref, w1_ref, b1_ref, w2_ref, b2_ref, g_ref, out_ref,
              acc_ref):
    h = pl.program_id(1)
    r = pl.program_id(2)
    xb = xe_ref[...]
    hid = jnp.dot(xb, w1_ref[0], preferred_element_type=jnp.float32)
    hid = jnp.maximum(hid + b1_ref[0], 0.0)
    partial = jnp.dot(hid, w2_ref[0], preferred_element_type=jnp.float32)

    @pl.when(h == 0)
    def _():
        acc_ref[pl.ds(r * MB, MB), :] = partial

    @pl.when(h == NH - 1)
    def _():
        o = acc_ref[pl.ds(r * MB, MB), :] + partial + b2_ref[0]
        out_ref[...] = o * g_ref[0, 0].reshape(MB, 1)


def _mlp(xe, w1, b1r, w2, b2r, gate3):
    return pl.pallas_call(
        _mlp_body,
        grid=(E, NH, MR),
        in_specs=[
            pl.BlockSpec((MB, D), lambda e, h, r: (e * MR + r, 0)),
            pl.BlockSpec((1, D, HB), lambda e, h, r: (e, 0, h)),
            pl.BlockSpec((1, 1, HB), lambda e, h, r: (e, 0, h)),
            pl.BlockSpec((1, HB, D), lambda e, h, r: (e, h, 0)),
            pl.BlockSpec((1, 1, D), lambda e, h, r: (e, 0, 0)),
            pl.BlockSpec((1, 1, MB), lambda e, h, r: (e * MR + r, 0, 0)),
        ],
        out_specs=pl.BlockSpec((MB, D), lambda e, h, r: (e * MR + r, 0)),
        out_shape=jax.ShapeDtypeStruct((NSLOT, D), jnp.float32),
        scratch_shapes=[pltpu.VMEM((CAP, D), jnp.float32)],
    )(xe, w1, b1r, w2, b2r, gate3)


def kernel(x, router_w, router_b, w1, b1, w2, b2):
    x_flat = x.reshape(T, D)
    ti3, tv3, _cs, _cnt, aux = _router(x_flat, router_w,
                                       router_b.reshape(1, E))
    top_idx = ti3.reshape(T)
    top_vals = tv3.reshape(T)
    disp, gate, cnt16, comb_idx = _dispatch(top_idx, top_vals)
    xe = _row_gather(x_flat, disp.reshape(NSLOT), NSLOT)
    out_all = _mlp(xe, w1, b1.reshape(E, 1, H), w2, b2.reshape(E, 1, D),
                   gate.reshape(E * MR, 1, MB))
    out_pad = jnp.concatenate(
        [out_all, jnp.zeros((8, D), jnp.float32)], axis=0)
    y_flat = _row_gather(out_pad, comb_idx, T)
    return (y_flat.reshape(B, S, D), aux.reshape(()), cnt16[:, 0])


# speedup vs baseline: 1.0530x; 1.0127x over previous
import functools

import jax
import jax.numpy as jnp
from jax import lax
from jax.experimental import pallas as pl
from jax.experimental.pallas import tpu as pltpu
from jax.experimental.pallas import tpu_sc as plsc

B = 4
S = 2048
T = B * S
D = 1024
H = 4096
E = 8
CAP = int(1.25 * T / E)
NSLOT = E * CAP
PAD_ROW = NSLOT
AUX_COEF = 0.01

RT = 8
RB = T // RT

MB = 256
MR = CAP // MB
NH = 2
HB = H // NH


def _router_body(x_ref, rw_ref, rb_ref, ti_ref, tv_ref, cs_ref, cnt_ref,
                 aux_ref):
    i = pl.program_id(0)
    xb = x_ref[...]
    logits = jnp.dot(xb, rw_ref[...], preferred_element_type=jnp.float32)
    logits = logits + rb_ref[...]
    m = jnp.max(logits, axis=-1, keepdims=True)
    ex = jnp.exp(logits - m)
    gates = ex / jnp.sum(ex, axis=-1, keepdims=True)
    tv = jnp.max(gates, axis=-1)
    ti = jnp.argmax(gates, axis=-1).astype(jnp.int32)
    ti_ref[...] = ti.reshape(1, 1, RB)
    tv_ref[...] = tv.reshape(1, 1, RB)
    cs = jnp.sum(gates, axis=0).reshape(1, E)
    oh = (ti[:, None] == lax.broadcasted_iota(jnp.int32, (1, E), 1))
    cnt = jnp.sum(oh.astype(jnp.float32), axis=0).reshape(1, E)

    @pl.when(i == 0)
    def _():
        cs_ref[...] = cs
        cnt_ref[...] = cnt

    @pl.when(i > 0)
    def _():
        cs_ref[...] = cs_ref[...] + cs
        cnt_ref[...] = cnt_ref[...] + cnt

    @pl.when(i == RT - 1)
    def _():
        imp = cs_ref[...] / float(T)
        load = cnt_ref[...] / float(T)
        aux_ref[...] = jnp.sum(imp * load).reshape(1, 1) * (E * AUX_COEF)


def _router(x_flat, rw, rb2):
    return pl.pallas_call(
        _router_body,
        grid=(RT,),
        in_specs=[
            pl.BlockSpec((RB, D), lambda i: (i, 0)),
            pl.BlockSpec((D, E), lambda i: (0, 0)),
            pl.BlockSpec((1, E), lambda i: (0, 0)),
        ],
        out_specs=[
            pl.BlockSpec((1, 1, RB), lambda i: (i, 0, 0)),
            pl.BlockSpec((1, 1, RB), lambda i: (i, 0, 0)),
            pl.BlockSpec((1, E), lambda i: (0, 0)),
            pl.BlockSpec((1, E), lambda i: (0, 0)),
            pl.BlockSpec((1, 1), lambda i: (0, 0)),
        ],
        out_shape=[
            jax.ShapeDtypeStruct((RT, 1, RB), jnp.int32),
            jax.ShapeDtypeStruct((RT, 1, RB), jnp.float32),
            jax.ShapeDtypeStruct((1, E), jnp.float32),
            jax.ShapeDtypeStruct((1, E), jnp.float32),
            jax.ShapeDtypeStruct((1, 1), jnp.float32),
        ],
    )(x_flat, rw, rb2)


def _dispatch_body(ti_hbm, tv_hbm, disp_hbm, gate_hbm, cnt_hbm, comb_hbm,
                   idx_v, tv_v, inv_v, disp_v, gate_v, s16_v, buf_v, acc_v,
                   inv_sh):
    c = lax.axis_index("c")
    s = lax.axis_index("s")

    @pl.when(jnp.logical_and(c == 0, s < E))
    def _phase_a():
        e = s
        pltpu.sync_copy(ti_hbm, idx_v)
        pltpu.sync_copy(tv_hbm, tv_v)

        def tok_body(i, base):
            v = idx_v[pl.ds(i * 16, 16)]
            m = v == e
            mi = jnp.where(m, 1, 0)
            csum = plsc.cumsum(mi)
            ranks = base + csum - 1
            keep = jnp.logical_and(m, ranks < CAP)
            tok = lax.iota(jnp.int32, 16) + i * 16
            slotp1 = e * CAP + ranks + 1
            inv_v[pl.ds(i * 16, 16)] = jnp.where(keep, slotp1, 0)
            ranks_c = jnp.where(keep, ranks, 0)
            plsc.store_scatter(disp_v, [ranks_c], tok, mask=keep)
            return base + jnp.sum(mi)

        total = lax.fori_loop(0, T // 16, tok_body, jnp.int32(0))
        count = jnp.minimum(total, CAP)

        def gate_body(j, _):
            pos = lax.iota(jnp.int32, 16) + j * 16
            valid = pos < count
            idxs = jnp.where(valid, disp_v[pl.ds(j * 16, 16)], 0)
            g = plsc.load_gather(tv_v, [idxs])
            gate_v[pl.ds(j * 16, 16)] = jnp.where(valid, g, 0.0)
            disp_v[pl.ds(j * 16, 16)] = idxs
            return 0

        lax.fori_loop(0, CAP // 16, gate_body, 0)
        s16_v[...] = jnp.where(lax.iota(jnp.int32, 16) == 0, count, 0)
        pltpu.sync_copy(disp_v, disp_hbm.at[e])
        pltpu.sync_copy(gate_v, gate_hbm.at[e])
        pltpu.sync_copy(s16_v, cnt_hbm.at[e])
        pltpu.sync_copy(inv_v, inv_sh.at[e])

    plsc.subcore_barrier()

    @pl.when(c == 0)
    def _phase_b():
        t0 = s * (T // 16)
        nch = T // 16 // 16
        for e in range(E):
            pltpu.sync_copy(inv_sh.at[e, pl.ds(t0, T // 16)], buf_v)

            def add_body(k, _):
                chunk = buf_v[pl.ds(k * 16, 16)]
                if e == 0:
                    acc_v[pl.ds(k * 16, 16)] = chunk
                else:
                    acc_v[pl.ds(k * 16, 16)] = acc_v[pl.ds(k * 16, 16)] + chunk
                return 0

            lax.fori_loop(0, nch, add_body, 0)

        def fin_body(k, _):
            iv = acc_v[pl.ds(k * 16, 16)]
            buf_v[pl.ds(k * 16, 16)] = jnp.where(iv > 0, iv - 1, PAD_ROW)
            return 0

        lax.fori_loop(0, nch, fin_body, 0)
        pltpu.sync_copy(buf_v, comb_hbm.at[pl.ds(t0, T // 16)])


def _dispatch(top_idx, top_vals):
    mesh = plsc.VectorSubcoreMesh(core_axis_name="c", subcore_axis_name="s")
    fn = pl.kernel(
        _dispatch_body,
        mesh=mesh,
        compiler_params=pltpu.CompilerParams(needs_layout_passes=False),
        out_type=[
            jax.ShapeDtypeStruct((E, CAP), jnp.int32),
            jax.ShapeDtypeStruct((E, CAP), jnp.float32),
            jax.ShapeDtypeStruct((E, 16), jnp.int32),
            jax.ShapeDtypeStruct((T,), jnp.int32),
        ],
        scratch_types=[
            pltpu.VMEM((T,), jnp.int32),
            pltpu.VMEM((T,), jnp.float32),
            pltpu.VMEM((T,), jnp.int32),
            pltpu.VMEM((CAP,), jnp.int32),
            pltpu.VMEM((CAP,), jnp.float32),
            pltpu.VMEM((16,), jnp.int32),
            pltpu.VMEM((T // 16,), jnp.int32),
            pltpu.VMEM((T // 16,), jnp.int32),
            pltpu.MemorySpace.VMEM_SHARED((E, T), jnp.int32),
        ],
    )
    return fn(top_idx, top_vals)


def _row_gather_body(nrows, chunk, tbl_hbm, idx_hbm, out_hbm, idx_v, buf0,
                     buf1, gs0, gs1, ws0, ws1):
    c = lax.axis_index("c")
    s = lax.axis_index("s")
    wid = s * 2 + c
    base = wid * nrows
    pltpu.sync_copy(idx_hbm.at[pl.ds(base, nrows)], idx_v)
    n = nrows // chunk
    bufs = (buf0, buf1)
    gsems = (gs0, gs1)
    wsems = (ws0, ws1)

    def start_gather(j):
        return pltpu.async_copy(
            tbl_hbm.at[idx_v.at[pl.ds(j * chunk, chunk)]], bufs[j % 2],
            gsems[j % 2])

    def start_write(j):
        return pltpu.async_copy(
            bufs[j % 2], out_hbm.at[pl.ds(base + j * chunk, chunk)],
            wsems[j % 2])

    gh = {0: start_gather(0)}
    wh = {}
    for j in range(n):
        if j + 1 < n:
            if j - 1 in wh:
                wh[j - 1].wait()
            gh[j + 1] = start_gather(j + 1)
        gh[j].wait()
        wh[j] = start_write(j)
    wh[n - 2].wait()
    wh[n - 1].wait()


def _row_gather(tbl, idx, nrows_total):
    nrows = nrows_total // 32
    chunk = 32 if nrows % 32 == 0 else 16
    mesh = plsc.VectorSubcoreMesh(core_axis_name="c", subcore_axis_name="s")
    fn = pl.kernel(
        functools.partial(_row_gather_body, nrows, chunk),
        mesh=mesh,
        compiler_params=pltpu.CompilerParams(needs_layout_passes=False),
        out_type=jax.ShapeDtypeStruct((nrows_total, D), jnp.float32),
        scratch_types=[
            pltpu.VMEM((nrows,), jnp.int32),
            pltpu.VMEM((chunk, D), jnp.float32),
            pltpu.VMEM((chunk, D), jnp.float32),
            pltpu.SemaphoreType.DMA,
            pltpu.SemaphoreType.DMA,
            pltpu.SemaphoreType.DMA,
            pltpu.SemaphoreType.DMA,
        ],
    )
    return fn(tbl, idx)


def _mlp_body(xe_ref, w1_ref, b1_ref, w2_ref, b2_ref, g_ref, out_ref,
              acc_ref):
    h = pl.program_id(1)
    r = pl.program_id(2)
    xb = xe_ref[...]
    hid = jnp.dot(xb, w1_ref[0], preferred_element_type=jnp.float32)
    hid = jnp.maximum(hid + b1_ref[0], 0.0)
    partial = jnp.dot(hid, w2_ref[0], preferred_element_type=jnp.float32)

    @pl.when(h == 0)
    def _():
        acc_ref[pl.ds(r * MB, MB), :] = partial

    @pl.when(h == NH - 1)
    def _():
        o = acc_ref[pl.ds(r * MB, MB), :] + partial + b2_ref[0]
        out_ref[...] = o * g_ref[0, 0].reshape(MB, 1)


def _mlp(xe, w1, b1r, w2, b2r, gate3, e0, ne):
    return pl.pallas_call(
        _mlp_body,
        grid=(ne, NH, MR),
        in_specs=[
            pl.BlockSpec((MB, D), lambda e, h, r: (e * MR + r, 0)),
            pl.BlockSpec((1, D, HB), lambda e, h, r: (e0 + e, 0, h)),
            pl.BlockSpec((1, 1, HB), lambda e, h, r: (e0 + e, 0, h)),
            pl.BlockSpec((1, HB, D), lambda e, h, r: (e0 + e, h, 0)),
            pl.BlockSpec((1, 1, D), lambda e, h, r: (e0 + e, 0, 0)),
            pl.BlockSpec((1, 1, MB),
                         lambda e, h, r: ((e0 + e) * MR + r, 0, 0)),
        ],
        out_specs=pl.BlockSpec((MB, D), lambda e, h, r: (e * MR + r, 0)),
        out_shape=jax.ShapeDtypeStruct((ne * CAP, D), jnp.float32),
        scratch_shapes=[pltpu.VMEM((CAP, D), jnp.float32)],
    )(xe, w1, b1r, w2, b2r, gate3)


def kernel(x, router_w, router_b, w1, b1, w2, b2):
    x_flat = x.reshape(T, D)
    ti3, tv3, _cs, _cnt, aux = _router(x_flat, router_w,
                                       router_b.reshape(1, E))
    top_idx = ti3.reshape(T)
    top_vals = tv3.reshape(T)
    disp, gate, cnt16, comb_idx = _dispatch(top_idx, top_vals)
    disp_f = disp.reshape(NSLOT)
    b1r = b1.reshape(E, 1, H)
    b2r = b2.reshape(E, 1, D)
    gate3 = gate.reshape(E * MR, 1, MB)
    nck = 4
    epc = E // nck
    rows = epc * CAP
    outs = []
    for k in range(nck):
        xe_k = _row_gather(x_flat, disp_f[k * rows:(k + 1) * rows], rows)
        outs.append(_mlp(xe_k, w1, b1r, w2, b2r, gate3, k * epc, epc))
    out_pad = jnp.concatenate(outs + [jnp.zeros((8, D), jnp.float32)],
                              axis=0)
    y_flat = _row_gather(out_pad, comb_idx, T)
    return (y_flat.reshape(B, S, D), aux.reshape(()), cnt16[:, 0])


# speedup vs baseline: 1.0970x; 1.0418x over previous
import functools

import jax
import jax.numpy as jnp
from jax import lax
from jax.experimental import pallas as pl
from jax.experimental.pallas import tpu as pltpu
from jax.experimental.pallas import tpu_sc as plsc

B = 4
S = 2048
T = B * S
D = 1024
H = 4096
E = 8
CAP = int(1.25 * T / E)
NSLOT = E * CAP
PAD_ROW = NSLOT
AUX_COEF = 0.01

RT = 8
RB = T // RT

MB = 256
MR = CAP // MB
NH = 2
HB = H // NH


def _router_body(x_ref, rw_ref, rb_ref, ti_ref, tv_ref, cs_ref, cnt_ref,
                 aux_ref):
    i = pl.program_id(0)
    xb = x_ref[...]
    logits = jnp.dot(xb, rw_ref[...], preferred_element_type=jnp.float32)
    logits = logits + rb_ref[...]
    m = jnp.max(logits, axis=-1, keepdims=True)
    ex = jnp.exp(logits - m)
    gates = ex / jnp.sum(ex, axis=-1, keepdims=True)
    tv = jnp.max(gates, axis=-1)
    ti = jnp.argmax(gates, axis=-1).astype(jnp.int32)
    ti_ref[...] = ti.reshape(1, 1, RB)
    tv_ref[...] = tv.reshape(1, 1, RB)
    cs = jnp.sum(gates, axis=0).reshape(1, E)
    oh = (ti[:, None] == lax.broadcasted_iota(jnp.int32, (1, E), 1))
    cnt = jnp.sum(oh.astype(jnp.float32), axis=0).reshape(1, E)

    @pl.when(i == 0)
    def _():
        cs_ref[...] = cs
        cnt_ref[...] = cnt

    @pl.when(i > 0)
    def _():
        cs_ref[...] = cs_ref[...] + cs
        cnt_ref[...] = cnt_ref[...] + cnt

    @pl.when(i == RT - 1)
    def _():
        imp = cs_ref[...] / float(T)
        load = cnt_ref[...] / float(T)
        aux_ref[...] = jnp.sum(imp * load).reshape(1, 1) * (E * AUX_COEF)


def _router(x_flat, rw, rb2):
    return pl.pallas_call(
        _router_body,
        grid=(RT,),
        in_specs=[
            pl.BlockSpec((RB, D), lambda i: (i, 0)),
            pl.BlockSpec((D, E), lambda i: (0, 0)),
            pl.BlockSpec((1, E), lambda i: (0, 0)),
        ],
        out_specs=[
            pl.BlockSpec((1, 1, RB), lambda i: (i, 0, 0)),
            pl.BlockSpec((1, 1, RB), lambda i: (i, 0, 0)),
            pl.BlockSpec((1, E), lambda i: (0, 0)),
            pl.BlockSpec((1, E), lambda i: (0, 0)),
            pl.BlockSpec((1, 1), lambda i: (0, 0)),
        ],
        out_shape=[
            jax.ShapeDtypeStruct((RT, 1, RB), jnp.int32),
            jax.ShapeDtypeStruct((RT, 1, RB), jnp.float32),
            jax.ShapeDtypeStruct((1, E), jnp.float32),
            jax.ShapeDtypeStruct((1, E), jnp.float32),
            jax.ShapeDtypeStruct((1, 1), jnp.float32),
        ],
    )(x_flat, rw, rb2)


def _dispatch_body(ti_hbm, tv_hbm, disp_hbm, gate_hbm, cnt_hbm, comb_hbm,
                   idx_v, tv_v, inv_v, disp_v, gate_v, s16_v, buf_v, acc_v,
                   inv_sh):
    c = lax.axis_index("c")
    s = lax.axis_index("s")

    @pl.when(jnp.logical_and(c == 0, s < E))
    def _phase_a():
        e = s
        pltpu.sync_copy(ti_hbm, idx_v)
        pltpu.sync_copy(tv_hbm, tv_v)

        def tok_body(i, base):
            v = idx_v[pl.ds(i * 16, 16)]
            m = v == e
            mi = jnp.where(m, 1, 0)
            csum = plsc.cumsum(mi)
            ranks = base + csum - 1
            keep = jnp.logical_and(m, ranks < CAP)
            tok = lax.iota(jnp.int32, 16) + i * 16
            slotp1 = e * CAP + ranks + 1
            inv_v[pl.ds(i * 16, 16)] = jnp.where(keep, slotp1, 0)
            ranks_c = jnp.where(keep, ranks, 0)
            plsc.store_scatter(disp_v, [ranks_c], tok, mask=keep)
            return base + jnp.sum(mi)

        total = lax.fori_loop(0, T // 16, tok_body, jnp.int32(0))
        count = jnp.minimum(total, CAP)

        def gate_body(j, _):
            pos = lax.iota(jnp.int32, 16) + j * 16
            valid = pos < count
            idxs = jnp.where(valid, disp_v[pl.ds(j * 16, 16)], 0)
            g = plsc.load_gather(tv_v, [idxs])
            gate_v[pl.ds(j * 16, 16)] = jnp.where(valid, g, 0.0)
            disp_v[pl.ds(j * 16, 16)] = idxs
            return 0

        lax.fori_loop(0, CAP // 16, gate_body, 0)
        s16_v[...] = jnp.where(lax.iota(jnp.int32, 16) == 0, count, 0)
        pltpu.sync_copy(disp_v, disp_hbm.at[e])
        pltpu.sync_copy(gate_v, gate_hbm.at[e])
        pltpu.sync_copy(s16_v, cnt_hbm.at[e])
        pltpu.sync_copy(inv_v, inv_sh.at[e])

    plsc.subcore_barrier()

    @pl.when(c == 0)
    def _phase_b():
        t0 = s * (T // 16)
        nch = T // 16 // 16
        for e in range(E):
            pltpu.sync_copy(inv_sh.at[e, pl.ds(t0, T // 16)], buf_v)

            def add_body(k, _):
                chunk = buf_v[pl.ds(k * 16, 16)]
                if e == 0:
                    acc_v[pl.ds(k * 16, 16)] = chunk
                else:
                    acc_v[pl.ds(k * 16, 16)] = acc_v[pl.ds(k * 16, 16)] + chunk
                return 0

            lax.fori_loop(0, nch, add_body, 0)

        def fin_body(k, _):
            iv = acc_v[pl.ds(k * 16, 16)]
            buf_v[pl.ds(k * 16, 16)] = jnp.where(iv > 0, iv - 1, PAD_ROW)
            return 0

        lax.fori_loop(0, nch, fin_body, 0)
        pltpu.sync_copy(buf_v, comb_hbm.at[pl.ds(t0, T // 16)])


def _dispatch(top_idx, top_vals):
    mesh = plsc.VectorSubcoreMesh(core_axis_name="c", subcore_axis_name="s")
    fn = pl.kernel(
        _dispatch_body,
        mesh=mesh,
        compiler_params=pltpu.CompilerParams(needs_layout_passes=False),
        out_type=[
            jax.ShapeDtypeStruct((E, CAP), jnp.int32),
            jax.ShapeDtypeStruct((E, CAP), jnp.float32),
            jax.ShapeDtypeStruct((E, 16), jnp.int32),
            jax.ShapeDtypeStruct((T,), jnp.int32),
        ],
        scratch_types=[
            pltpu.VMEM((T,), jnp.int32),
            pltpu.VMEM((T,), jnp.float32),
            pltpu.VMEM((T,), jnp.int32),
            pltpu.VMEM((CAP,), jnp.int32),
            pltpu.VMEM((CAP,), jnp.float32),
            pltpu.VMEM((16,), jnp.int32),
            pltpu.VMEM((T // 16,), jnp.int32),
            pltpu.VMEM((T // 16,), jnp.int32),
            pltpu.MemorySpace.VMEM_SHARED((E, T), jnp.int32),
        ],
    )
    return fn(top_idx, top_vals)


def _row_gather_body(nrows, chunk, tbl_hbm, idx_hbm, out_hbm, idx_v, buf0,
                     buf1, gs0, gs1, ws0, ws1):
    c = lax.axis_index("c")
    s = lax.axis_index("s")
    wid = s * 2 + c
    base = wid * nrows
    pltpu.sync_copy(idx_hbm.at[pl.ds(base, nrows)], idx_v)
    n = nrows // chunk
    bufs = (buf0, buf1)
    gsems = (gs0, gs1)
    wsems = (ws0, ws1)

    def start_gather(j):
        return pltpu.async_copy(
            tbl_hbm.at[idx_v.at[pl.ds(j * chunk, chunk)]], bufs[j % 2],
            gsems[j % 2])

    def start_write(j):
        return pltpu.async_copy(
            bufs[j % 2], out_hbm.at[pl.ds(base + j * chunk, chunk)],
            wsems[j % 2])

    gh = {0: start_gather(0)}
    wh = {}
    for j in range(n):
        if j + 1 < n:
            if j - 1 in wh:
                wh[j - 1].wait()
            gh[j + 1] = start_gather(j + 1)
        gh[j].wait()
        wh[j] = start_write(j)
    wh[n - 2].wait()
    wh[n - 1].wait()


def _row_gather(tbl, idx, nrows_total):
    nrows = nrows_total // 32
    chunk = 32 if nrows % 32 == 0 else 16
    mesh = plsc.VectorSubcoreMesh(core_axis_name="c", subcore_axis_name="s")
    fn = pl.kernel(
        functools.partial(_row_gather_body, nrows, chunk),
        mesh=mesh,
        compiler_params=pltpu.CompilerParams(needs_layout_passes=False),
        out_type=jax.ShapeDtypeStruct((nrows_total, D), jnp.float32),
        scratch_types=[
            pltpu.VMEM((nrows,), jnp.int32),
            pltpu.VMEM((chunk, D), jnp.float32),
            pltpu.VMEM((chunk, D), jnp.float32),
            pltpu.SemaphoreType.DMA,
            pltpu.SemaphoreType.DMA,
            pltpu.SemaphoreType.DMA,
            pltpu.SemaphoreType.DMA,
        ],
    )
    return fn(tbl, idx)


def _mlp_body(e0, cnt_ref, xe_ref, w1_ref, b1_ref, w2_ref, b2_ref, g_ref,
              buf_ref, out_ref, acc_ref):
    del buf_ref
    e = pl.program_id(0)
    h = pl.program_id(1)
    r = pl.program_id(2)

    @pl.when(r * MB < cnt_ref[e0 + e])
    def _():
        xb = xe_ref[...]
        hid = jnp.dot(xb, w1_ref[0], preferred_element_type=jnp.float32)
        hid = jnp.maximum(hid + b1_ref[0], 0.0)
        partial = jnp.dot(hid, w2_ref[0], preferred_element_type=jnp.float32)

        @pl.when(h == 0)
        def _():
            acc_ref[pl.ds(r * MB, MB), :] = partial

        @pl.when(h == NH - 1)
        def _():
            o = acc_ref[pl.ds(r * MB, MB), :] + partial + b2_ref[0]
            out_ref[...] = o * g_ref[0, 0].reshape(MB, 1)


def _mlp(cnt, xe, w1, b1r, w2, b2r, gate3, buf, e0, ne):
    return pl.pallas_call(
        functools.partial(_mlp_body, e0),
        grid_spec=pltpu.PrefetchScalarGridSpec(
            num_scalar_prefetch=1,
            grid=(ne, NH, MR),
            in_specs=[
                pl.BlockSpec((MB, D), lambda e, h, r, c: (e * MR + r, 0)),
                pl.BlockSpec((1, D, HB), lambda e, h, r, c: (e0 + e, 0, h)),
                pl.BlockSpec((1, 1, HB), lambda e, h, r, c: (e0 + e, 0, h)),
                pl.BlockSpec((1, HB, D), lambda e, h, r, c: (e0 + e, h, 0)),
                pl.BlockSpec((1, 1, D), lambda e, h, r, c: (e0 + e, 0, 0)),
                pl.BlockSpec((1, 1, MB),
                             lambda e, h, r, c: ((e0 + e) * MR + r, 0, 0)),
                pl.BlockSpec(memory_space=pl.ANY),
            ],
            out_specs=pl.BlockSpec(
                (MB, D), lambda e, h, r, c: ((e0 + e) * MR + r, 0)),
            scratch_shapes=[pltpu.VMEM((CAP, D), jnp.float32)],
        ),
        out_shape=jax.ShapeDtypeStruct((NSLOT + 8, D), jnp.float32),
        input_output_aliases={7: 0},
    )(cnt, xe, w1, b1r, w2, b2r, gate3, buf)


def kernel(x, router_w, router_b, w1, b1, w2, b2):
    x_flat = x.reshape(T, D)
    ti3, tv3, _cs, _cnt, aux = _router(x_flat, router_w,
                                       router_b.reshape(1, E))
    top_idx = ti3.reshape(T)
    top_vals = tv3.reshape(T)
    disp, gate, cnt16, comb_idx = _dispatch(top_idx, top_vals)
    disp_f = disp.reshape(NSLOT)
    b1r = b1.reshape(E, 1, H)
    b2r = b2.reshape(E, 1, D)
    gate3 = gate.reshape(E * MR, 1, MB)
    nck = 4
    epc = E // nck
    rows = epc * CAP
    cnt = cnt16[:, 0]
    out_pad = jnp.zeros((NSLOT + 8, D), jnp.float32)
    for k in range(nck):
        xe_k = _row_gather(x_flat, disp_f[k * rows:(k + 1) * rows], rows)
        out_pad = _mlp(cnt, xe_k, w1, b1r, w2, b2r, gate3, out_pad,
                       k * epc, epc)
    y_flat = _row_gather(out_pad, comb_idx, T)
    return (y_flat.reshape(B, S, D), aux.reshape(()), cnt16[:, 0])
